# baseline (speedup 1.0000x reference)
import jax
import jax.numpy as jnp
from jax.experimental import pallas as pl

B, S, D, H = 1, 2048, 1024, 16
E, K, HID = 4, 2, 4096
DH = D // H
BT = 256
HB = 512
NT = S // BT
NHB = HID // HB

_f32 = jnp.float32
_bf16 = jnp.bfloat16


def _dot(a, b, trans_b=False, prec=None):
    dims = (((1,), (1 if trans_b else 0,)), ((), ()))
    return jax.lax.dot_general(a, b, dims, preferred_element_type=_f32,
                               precision=prec)


_HI = jax.lax.Precision.HIGHEST


def _qkv_kern(x_ref, g_ref, b_ref, w_ref, bias_ref, o_ref):
    x = x_ref[...]
    m = jnp.mean(x, axis=-1, keepdims=True)
    v = jnp.mean((x - m) ** 2, axis=-1, keepdims=True)
    h = (x - m) / jnp.sqrt(v + 1e-5) * g_ref[...] + b_ref[...]
    o_ref[...] = _dot(h, w_ref[...], prec=_HI) + bias_ref[...]


def _attn_kern(q_ref, k_ref, v_ref, o_ref):
    t = pl.program_id(1)
    q = q_ref[0]
    k = k_ref[0]
    s = _dot(q, k, trans_b=True, prec=_HI) * (1.0 / 8.0)
    row = t * BT + jax.lax.broadcasted_iota(jnp.int32, (BT, S), 0)
    col = jax.lax.broadcasted_iota(jnp.int32, (BT, S), 1)
    s = jnp.where(col <= row, s, -1e9)
    m = jnp.max(s, axis=-1, keepdims=True)
    e = jnp.exp(s - m)
    p = e / jnp.sum(e, axis=-1, keepdims=True)
    o_ref[0] = _dot(p, v_ref[0], prec=_HI)


def _post_kern(ctx_ref, wo_ref, bo_ref, x_ref, g_ref, b_ref, wg_ref,
               x1_ref, tok_ref, lg_ref):
    ao = _dot(ctx_ref[...], wo_ref[...], prec=_HI) + bo_ref[...]
    x1 = x_ref[...] + ao
    x1_ref[...] = x1
    m = jnp.mean(x1, axis=-1, keepdims=True)
    v = jnp.mean((x1 - m) ** 2, axis=-1, keepdims=True)
    tok = (x1 - m) / jnp.sqrt(v + 1e-5) * g_ref[...] + b_ref[...]
    tok_ref[...] = tok
    lg_ref[...] = _dot(tok, wg_ref[...], prec=_HI)


def _route_kern(lg_ref, w_ref, aux_ref):
    lg = lg_ref[...]
    m = jnp.max(lg, axis=-1, keepdims=True)
    ex = jnp.exp(lg - m)
    p = ex / jnp.sum(ex, axis=-1, keepdims=True)
    iota = jax.lax.broadcasted_iota(jnp.int32, (S, E), 1)
    m1 = jnp.max(p, axis=-1, keepdims=True)
    i1 = jnp.min(jnp.where(p == m1, iota, E), axis=-1, keepdims=True)
    pm = jnp.where(iota == i1, -1.0, p)
    m2 = jnp.max(pm, axis=-1, keepdims=True)
    i2 = jnp.min(jnp.where(pm == m2, iota, E), axis=-1, keepdims=True)
    wsum = m1 + m2
    w = jnp.where(iota == i1, m1 / wsum, 0.0) + jnp.where(iota == i2, m2 / wsum, 0.0)
    w_ref[...] = w
    sel = jnp.logical_or(iota == i1, iota == i2)
    fi = jnp.sum(sel.astype(_f32), axis=0, keepdims=True) / (S * K)
    Pi = jnp.mean(p, axis=0, keepdims=True)
    aux_ref[...] = 0.01 * E * jnp.sum(fi * Pi, axis=-1, keepdims=True)


def _ffn_kern(tok_ref, x1_ref, w_ref, we1_ref, be1_ref, we2_ref, be2_ref,
              ws1_ref, bs1_ref, ws2_ref, bs2_ref, o_ref):
    hb = pl.program_id(0)
    t = pl.program_id(1)
    rows = pl.ds(t * BT, BT)
    tokb = tok_ref[rows, :].astype(_bf16)
    eh = jax.nn.gelu(_dot(tokb, ws1_ref[...]) + bs1_ref[...])
    acc = _dot(eh.astype(_bf16), ws2_ref[...])
    for e in range(E):
        ehe = jax.nn.gelu(_dot(tokb, we1_ref[e]) + be1_ref[e:e + 1, :])
        pe = _dot(ehe.astype(_bf16), we2_ref[e])
        acc += w_ref[rows, e:e + 1] * pe

    @pl.when(hb == 0)
    def _():
        base = x1_ref[rows, :] + bs2_ref[...]
        for e in range(E):
            base += w_ref[rows, e:e + 1] * be2_ref[e:e + 1, :]
        o_ref[rows, :] = base + acc

    @pl.when(hb != 0)
    def _():
        o_ref[rows, :] += acc


def kernel(x, ln1_g, ln1_b, Wqkv, bqkv, Wo, bo, ln2_g, ln2_b, Wg,
           We1, be1, We2, be2, Ws1, bs1, Ws2, bs2):
    x2 = x.reshape(S, D)
    row1 = lambda a: a.reshape(1, -1)

    qkv = pl.pallas_call(
        _qkv_kern,
        grid=(NT,),
        in_specs=[
            pl.BlockSpec((BT, D), lambda i: (i, 0)),
            pl.BlockSpec((1, D), lambda i: (0, 0)),
            pl.BlockSpec((1, D), lambda i: (0, 0)),
            pl.BlockSpec((D, 3 * D), lambda i: (0, 0)),
            pl.BlockSpec((1, 3 * D), lambda i: (0, 0)),
        ],
        out_specs=pl.BlockSpec((BT, 3 * D), lambda i: (i, 0)),
        out_shape=jax.ShapeDtypeStruct((S, 3 * D), _f32),
    )(x2, row1(ln1_g), row1(ln1_b), Wqkv, row1(bqkv))

    q3 = qkv[:, :D].reshape(S, H, DH).transpose(1, 0, 2)
    k3 = qkv[:, D:2 * D].reshape(S, H, DH).transpose(1, 0, 2)
    v3 = qkv[:, 2 * D:].reshape(S, H, DH).transpose(1, 0, 2)

    ctx3 = pl.pallas_call(
        _attn_kern,
        grid=(H, NT),
        in_specs=[
            pl.BlockSpec((1, BT, DH), lambda h, t: (h, t, 0)),
            pl.BlockSpec((1, S, DH), lambda h, t: (h, 0, 0)),
            pl.BlockSpec((1, S, DH), lambda h, t: (h, 0, 0)),
        ],
        out_specs=pl.BlockSpec((1, BT, DH), lambda h, t: (h, t, 0)),
        out_shape=jax.ShapeDtypeStruct((H, S, DH), _f32),
    )(q3, k3, v3)
    ctx = ctx3.transpose(1, 0, 2).reshape(S, D)

    x1, tok, logits = pl.pallas_call(
        _post_kern,
        grid=(NT,),
        in_specs=[
            pl.BlockSpec((BT, D), lambda i: (i, 0)),
            pl.BlockSpec((D, D), lambda i: (0, 0)),
            pl.BlockSpec((1, D), lambda i: (0, 0)),
            pl.BlockSpec((BT, D), lambda i: (i, 0)),
            pl.BlockSpec((1, D), lambda i: (0, 0)),
            pl.BlockSpec((1, D), lambda i: (0, 0)),
            pl.BlockSpec((D, E), lambda i: (0, 0)),
        ],
        out_specs=[
            pl.BlockSpec((BT, D), lambda i: (i, 0)),
            pl.BlockSpec((BT, D), lambda i: (i, 0)),
            pl.BlockSpec((BT, E), lambda i: (i, 0)),
        ],
        out_shape=[
            jax.ShapeDtypeStruct((S, D), _f32),
            jax.ShapeDtypeStruct((S, D), _f32),
            jax.ShapeDtypeStruct((S, E), _f32),
        ],
    )(ctx, Wo, row1(bo), x2, row1(ln2_g), row1(ln2_b), Wg)

    w, aux = pl.pallas_call(
        _route_kern,
        grid=(1,),
        in_specs=[pl.BlockSpec((S, E), lambda i: (0, 0))],
        out_specs=[
            pl.BlockSpec((S, E), lambda i: (0, 0)),
            pl.BlockSpec((1, 1), lambda i: (0, 0)),
        ],
        out_shape=[
            jax.ShapeDtypeStruct((S, E), _f32),
            jax.ShapeDtypeStruct((1, 1), _f32),
        ],
    )(logits)

    out = pl.pallas_call(
        _ffn_kern,
        grid=(NHB, NT),
        in_specs=[
            pl.BlockSpec((S, D), lambda hb, t: (0, 0)),
            pl.BlockSpec((S, D), lambda hb, t: (0, 0)),
            pl.BlockSpec((S, E), lambda hb, t: (0, 0)),
            pl.BlockSpec((E, D, HB), lambda hb, t: (0, 0, hb)),
            pl.BlockSpec((E, HB), lambda hb, t: (0, hb)),
            pl.BlockSpec((E, HB, D), lambda hb, t: (0, hb, 0)),
            pl.BlockSpec((E, D), lambda hb, t: (0, 0)),
            pl.BlockSpec((D, HB), lambda hb, t: (0, hb)),
            pl.BlockSpec((1, HB), lambda hb, t: (0, hb)),
            pl.BlockSpec((HB, D), lambda hb, t: (hb, 0)),
            pl.BlockSpec((1, D), lambda hb, t: (0, 0)),
        ],
        out_specs=pl.BlockSpec((S, D), lambda hb, t: (0, 0)),
        out_shape=jax.ShapeDtypeStruct((S, D), _f32),
    )(tok, x1, w, We1.astype(_bf16), be1, We2.astype(_bf16), be2,
      Ws1.astype(_bf16), row1(bs1), Ws2.astype(_bf16), row1(bs2))

    return (aux[0, 0], out.reshape(B, S, D))


# speedup vs baseline: 1.9768x; 1.9768x over previous
import jax
import jax.numpy as jnp
from jax.experimental import pallas as pl
from jax.experimental.pallas import tpu as pltpu

B, S, D, H = 1, 2048, 1024, 16
E, K, HID = 4, 2, 4096
DH = D // H
BT = 256
HB = 512
NT = S // BT
NHB = HID // HB

_f32 = jnp.float32
_bf16 = jnp.bfloat16


def _dot(a, b, trans_b=False, prec=None):
    dims = (((1,), (1 if trans_b else 0,)), ((), ()))
    return jax.lax.dot_general(a, b, dims, preferred_element_type=_f32,
                               precision=prec)


_HI = jax.lax.Precision.HIGHEST


def _qkv_kern(x_ref, g_ref, b_ref, w_ref, bias_ref, o_ref):
    x = x_ref[...]
    m = jnp.mean(x, axis=-1, keepdims=True)
    v = jnp.mean((x - m) ** 2, axis=-1, keepdims=True)
    h = (x - m) / jnp.sqrt(v + 1e-5) * g_ref[...] + b_ref[...]
    o_ref[...] = _dot(h.astype(_bf16), w_ref[...]) + bias_ref[...]


def _attn_kern(q_ref, k_ref, v_ref, o_ref, s_scr):
    t = pl.program_id(1)
    q = q_ref[0].astype(_bf16)

    def fill(kb, c):
        @pl.when(kb <= t)
        def _():
            kblk = k_ref[0, pl.ds(kb * BT, BT), :].astype(_bf16)
            s = _dot(q, kblk, trans_b=True) * (1.0 / 8.0)
            row = t * BT + jax.lax.broadcasted_iota(jnp.int32, (BT, BT), 0)
            col = kb * BT + jax.lax.broadcasted_iota(jnp.int32, (BT, BT), 1)
            s_scr[:, pl.ds(kb * BT, BT)] = jnp.where(col <= row, s, -1e9)

        @pl.when(kb > t)
        def _():
            s_scr[:, pl.ds(kb * BT, BT)] = jnp.full((BT, BT), -1e9, _f32)

        return c

    jax.lax.fori_loop(0, NT, fill, 0)
    s = s_scr[...]
    m = jnp.max(s, axis=-1, keepdims=True)
    e = jnp.exp(s - m)
    p = e / jnp.sum(e, axis=-1, keepdims=True)
    o_ref[0] = _dot(p.astype(_bf16), v_ref[0].astype(_bf16))


def _post_kern(ctx_ref, wo_ref, bo_ref, x_ref, g_ref, b_ref, wg_ref,
               x1_ref, tok_ref, lg_ref):
    ao = _dot(ctx_ref[...].astype(_bf16), wo_ref[...]) + bo_ref[...]
    x1 = x_ref[...] + ao
    x1_ref[...] = x1
    m = jnp.mean(x1, axis=-1, keepdims=True)
    v = jnp.mean((x1 - m) ** 2, axis=-1, keepdims=True)
    tok = (x1 - m) / jnp.sqrt(v + 1e-5) * g_ref[...] + b_ref[...]
    tok_ref[...] = tok
    lg_ref[...] = _dot(tok.astype(_bf16), wg_ref[...].astype(_bf16))


def _route_kern(lg_ref, w_ref, aux_ref):
    lg = lg_ref[...]
    m = jnp.max(lg, axis=-1, keepdims=True)
    ex = jnp.exp(lg - m)
    p = ex / jnp.sum(ex, axis=-1, keepdims=True)
    iota = jax.lax.broadcasted_iota(jnp.int32, (S, E), 1)
    m1 = jnp.max(p, axis=-1, keepdims=True)
    i1 = jnp.min(jnp.where(p == m1, iota, E), axis=-1, keepdims=True)
    pm = jnp.where(iota == i1, -1.0, p)
    m2 = jnp.max(pm, axis=-1, keepdims=True)
    i2 = jnp.min(jnp.where(pm == m2, iota, E), axis=-1, keepdims=True)
    wsum = m1 + m2
    w = jnp.where(iota == i1, m1 / wsum, 0.0) + jnp.where(iota == i2, m2 / wsum, 0.0)
    w_ref[...] = w
    sel = jnp.logical_or(iota == i1, iota == i2)
    fi = jnp.sum(sel.astype(_f32), axis=0, keepdims=True) / (S * K)
    Pi = jnp.mean(p, axis=0, keepdims=True)
    aux_ref[...] = 0.01 * E * jnp.sum(fi * Pi, axis=-1, keepdims=True)


def _ffn_kern(tok_ref, x1_ref, w_ref, we1_ref, be1_ref, we2_ref, be2_ref,
              ws1_ref, bs1_ref, ws2_ref, bs2_ref, o_ref):
    hb = pl.program_id(0)
    t = pl.program_id(1)
    rows = pl.ds(t * BT, BT)
    tokb = tok_ref[rows, :].astype(_bf16)
    eh = jax.nn.gelu(_dot(tokb, ws1_ref[...]) + bs1_ref[...])
    acc = _dot(eh.astype(_bf16), ws2_ref[...])
    for e in range(E):
        ehe = jax.nn.gelu(_dot(tokb, we1_ref[e]) + be1_ref[e:e + 1, :])
        pe = _dot(ehe.astype(_bf16), we2_ref[e])
        acc += w_ref[rows, e:e + 1] * pe

    @pl.when(hb == 0)
    def _():
        base = x1_ref[rows, :] + bs2_ref[...]
        for e in range(E):
            base += w_ref[rows, e:e + 1] * be2_ref[e:e + 1, :]
        o_ref[rows, :] = base + acc

    @pl.when(hb != 0)
    def _():
        o_ref[rows, :] += acc


def kernel(x, ln1_g, ln1_b, Wqkv, bqkv, Wo, bo, ln2_g, ln2_b, Wg,
           We1, be1, We2, be2, Ws1, bs1, Ws2, bs2):
    x2 = x.reshape(S, D)
    row1 = lambda a: a.reshape(1, -1)

    qkv = pl.pallas_call(
        _qkv_kern,
        grid=(NT,),
        in_specs=[
            pl.BlockSpec((BT, D), lambda i: (i, 0)),
            pl.BlockSpec((1, D), lambda i: (0, 0)),
            pl.BlockSpec((1, D), lambda i: (0, 0)),
            pl.BlockSpec((D, 3 * D), lambda i: (0, 0)),
            pl.BlockSpec((1, 3 * D), lambda i: (0, 0)),
        ],
        out_specs=pl.BlockSpec((BT, 3 * D), lambda i: (i, 0)),
        out_shape=jax.ShapeDtypeStruct((S, 3 * D), _f32),
    )(x2, row1(ln1_g), row1(ln1_b), Wqkv.astype(_bf16), row1(bqkv))

    q3 = qkv[:, :D].reshape(S, H, DH).transpose(1, 0, 2)
    k3 = qkv[:, D:2 * D].reshape(S, H, DH).transpose(1, 0, 2)
    v3 = qkv[:, 2 * D:].reshape(S, H, DH).transpose(1, 0, 2)

    ctx3 = pl.pallas_call(
        _attn_kern,
        grid=(H, NT),
        in_specs=[
            pl.BlockSpec((1, BT, DH), lambda h, t: (h, t, 0)),
            pl.BlockSpec((1, S, DH), lambda h, t: (h, 0, 0)),
            pl.BlockSpec((1, S, DH), lambda h, t: (h, 0, 0)),
        ],
        out_specs=pl.BlockSpec((1, BT, DH), lambda h, t: (h, t, 0)),
        out_shape=jax.ShapeDtypeStruct((H, S, DH), _f32),
        scratch_shapes=[pltpu.VMEM((BT, S), _f32)],
    )(q3, k3, v3)
    ctx = ctx3.transpose(1, 0, 2).reshape(S, D)

    x1, tok, logits = pl.pallas_call(
        _post_kern,
        grid=(NT,),
        in_specs=[
            pl.BlockSpec((BT, D), lambda i: (i, 0)),
            pl.BlockSpec((D, D), lambda i: (0, 0)),
            pl.BlockSpec((1, D), lambda i: (0, 0)),
            pl.BlockSpec((BT, D), lambda i: (i, 0)),
            pl.BlockSpec((1, D), lambda i: (0, 0)),
            pl.BlockSpec((1, D), lambda i: (0, 0)),
            pl.BlockSpec((D, E), lambda i: (0, 0)),
        ],
        out_specs=[
            pl.BlockSpec((BT, D), lambda i: (i, 0)),
            pl.BlockSpec((BT, D), lambda i: (i, 0)),
            pl.BlockSpec((BT, E), lambda i: (i, 0)),
        ],
        out_shape=[
            jax.ShapeDtypeStruct((S, D), _f32),
            jax.ShapeDtypeStruct((S, D), _f32),
            jax.ShapeDtypeStruct((S, E), _f32),
        ],
    )(ctx, Wo.astype(_bf16), row1(bo), x2, row1(ln2_g), row1(ln2_b), Wg)

    w, aux = pl.pallas_call(
        _route_kern,
        grid=(1,),
        in_specs=[pl.BlockSpec((S, E), lambda i: (0, 0))],
        out_specs=[
            pl.BlockSpec((S, E), lambda i: (0, 0)),
            pl.BlockSpec((1, 1), lambda i: (0, 0)),
        ],
        out_shape=[
            jax.ShapeDtypeStruct((S, E), _f32),
            jax.ShapeDtypeStruct((1, 1), _f32),
        ],
    )(logits)

    out = pl.pallas_call(
        _ffn_kern,
        grid=(NHB, NT),
        in_specs=[
            pl.BlockSpec((S, D), lambda hb, t: (0, 0)),
            pl.BlockSpec((S, D), lambda hb, t: (0, 0)),
            pl.BlockSpec((S, E), lambda hb, t: (0, 0)),
            pl.BlockSpec((E, D, HB), lambda hb, t: (0, 0, hb)),
            pl.BlockSpec((E, HB), lambda hb, t: (0, hb)),
            pl.BlockSpec((E, HB, D), lambda hb, t: (0, hb, 0)),
            pl.BlockSpec((E, D), lambda hb, t: (0, 0)),
            pl.BlockSpec((D, HB), lambda hb, t: (0, hb)),
            pl.BlockSpec((1, HB), lambda hb, t: (0, hb)),
            pl.BlockSpec((HB, D), lambda hb, t: (hb, 0)),
            pl.BlockSpec((1, D), lambda hb, t: (0, 0)),
        ],
        out_specs=pl.BlockSpec((S, D), lambda hb, t: (0, 0)),
        out_shape=jax.ShapeDtypeStruct((S, D), _f32),
    )(tok, x1, w, We1.astype(_bf16), be1, We2.astype(_bf16), be2,
      Ws1.astype(_bf16), row1(bs1), Ws2.astype(_bf16), row1(bs2))

    return (aux[0, 0], out.reshape(B, S, D))
